# baseline (speedup 1.0000x reference)
import jax
import jax.numpy as jnp
from jax import lax
from jax.experimental import pallas as pl
from jax.experimental.pallas import tpu as pltpu
from jax.experimental.pallas import tpu_sc as plsc

E = 8
D = 1024
T = 4096
K = 2
N = T * K
BM = 256
NBLK = (N + E * (BM - 1) + BM - 1) // BM
NSLOT = NBLK * BM
TB = 512

NC = 2
NS = 16
NW = NC * NS
CH = N // NW
NV = CH // 16
GCH = 32
NGC = CH // GCH
EOB_PAD = 48


def _routing_body(x_ref, gw_ref, idx_ref, w_ref):
    logits = jax.lax.dot_general(
        x_ref[...], gw_ref[...], (((1,), (1,)), ((), ())),
        preferred_element_type=jnp.float32)
    cols = jax.lax.broadcasted_iota(jnp.int32, (TB, E), 1)
    big = jnp.int32(E)
    m0 = jnp.max(logits, axis=1, keepdims=True)
    e0 = jnp.min(jnp.where(logits == m0, cols, big), axis=1, keepdims=True)
    neg = jnp.where(cols == e0, -jnp.inf, logits)
    m1 = jnp.max(neg, axis=1, keepdims=True)
    e1 = jnp.min(jnp.where(neg == m1, cols, big), axis=1, keepdims=True)
    t = jnp.exp(m1 - m0)
    w0 = 1.0 / (1.0 + t)
    w1 = 1.0 - w0
    idx_ref[...] = jnp.concatenate([e0, e1], axis=1)
    w_ref[...] = jnp.concatenate([w0, w1], axis=1)


def _routing(x, gate_w):
    return pl.pallas_call(
        _routing_body,
        grid=(T // TB,),
        in_specs=[
            pl.BlockSpec((TB, D), lambda i: (i, 0)),
            pl.BlockSpec((E, D), lambda i: (0, 0)),
        ],
        out_specs=[
            pl.BlockSpec((TB, K), lambda i: (i, 0)),
            pl.BlockSpec((TB, K), lambda i: (i, 0)),
        ],
        out_shape=[
            jax.ShapeDtypeStruct((T, K), jnp.int32),
            jax.ShapeDtypeStruct((T, K), jnp.float32),
        ],
    )(x, gate_w)


def _count_body(idx_hbm, counts_hbm, ranks_hbm, eloc_v, rank_v, cnt_v):
    cid = lax.axis_index("c")
    sid = lax.axis_index("s")
    wid = sid * NC + cid
    base = wid * CH
    lanes = lax.iota(jnp.int32, 16)

    pltpu.sync_copy(idx_hbm.at[pl.ds(base, CH)], eloc_v)
    c = [jnp.int32(0)] * E
    for i in range(NV):
        v = eloc_v[pl.ds(i * 16, 16)]
        d = jnp.zeros((16,), jnp.int32)
        for e in range(E):
            m = v == e
            mi = m.astype(jnp.int32)
            pre = plsc.cumsum(mi)
            d = jnp.where(m, c[e] + pre - 1, d)
            c[e] = c[e] + jnp.sum(mi)
        rank_v[pl.ds(i * 16, 16)] = d
    cvec = jnp.zeros((16,), jnp.int32)
    for e in range(E):
        cvec = jnp.where(lanes == e, c[e], cvec)
    cnt_v[...] = cvec
    pltpu.sync_copy(cnt_v, counts_hbm.at[wid])
    pltpu.sync_copy(rank_v, ranks_hbm.at[pl.ds(base, CH)])


def _dest_body(x_hbm, idx_hbm, counts_hbm, ranks_hbm,
               dest_hbm, xs_hbm, eob_hbm,
               eloc_v, rank_v, dstrow_v, dsc_v, tok_v, cnt_v, allcnt_v,
               eob_v, rows_v, rows2_v, gsem0, gsem1, ssem0, ssem1):
    cid = lax.axis_index("c")
    sid = lax.axis_index("s")
    wid = sid * NC + cid
    base = wid * CH
    lanes = lax.iota(jnp.int32, 16)

    pltpu.sync_copy(idx_hbm.at[pl.ds(base, CH)], eloc_v)
    pltpu.sync_copy(ranks_hbm.at[pl.ds(base, CH)], rank_v)
    pltpu.sync_copy(counts_hbm, allcnt_v)
    prior = jnp.zeros((16,), jnp.int32)
    gc = jnp.zeros((16,), jnp.int32)
    wid_v = jnp.full((16,), wid, jnp.int32)
    for w in range(NW):
        row = allcnt_v[w]
        gc = gc + row
        prior = prior + jnp.where(wid_v > w, row, 0)
    blocks = (gc + (BM - 1)) // BM
    psum = plsc.cumsum(blocks * BM)
    table = (psum - blocks * BM) + prior
    cnt_v[...] = table
    for i in range(NV):
        v = eloc_v[pl.ds(i * 16, 16)]
        b = plsc.load_gather(cnt_v, [v])
        dv = b + rank_v[pl.ds(i * 16, 16)]
        dstrow_v[(i * 16) // 128, pl.ds((i * 16) % 128, 16)] = dv
        nvc = GCH // 16
        dsc_v[i // nvc, pl.ds((i % nvc) * 16, 16)] = dv
        tok_v[i // nvc, pl.ds((i % nvc) * 16, 16)] = (base + i * 16 + lanes) // K
    for j in range(CH // 128):
        pltpu.sync_copy(dstrow_v.at[j], dest_hbm.at[wid * (CH // 128) + j])
    bufs = (rows_v, rows2_v)
    gsems = (gsem0, gsem1)
    ssems = (ssem0, ssem1)
    g_next = pltpu.async_copy(x_hbm.at[tok_v.at[0]], bufs[0], gsems[0])
    sc_h = [None, None]
    for cch in range(NGC):
        b = cch % 2
        g_next.wait()
        sc_h[b] = pltpu.async_copy(bufs[b], xs_hbm.at[dsc_v.at[cch]], ssems[b])
        if cch + 1 < NGC:
            nb = (cch + 1) % 2
            if sc_h[nb] is not None:
                sc_h[nb].wait()
            g_next = pltpu.async_copy(x_hbm.at[tok_v.at[cch + 1]],
                                      bufs[nb], gsems[nb])
    sc_h[0].wait()
    sc_h[1].wait()

    @pl.when(wid == 0)
    def _eob():
        cbv = plsc.cumsum(blocks)
        for r in range(EOB_PAD // 16):
            bi = lanes + 16 * r
            acc = jnp.zeros((16,), jnp.int32)
            for e in range(E - 1):
                acc = acc + (bi >= cbv[e]).astype(jnp.int32)
            if r == EOB_PAD // 16 - 1:
                acc = jnp.where(lanes == 15, cbv[E - 1], acc)
            eob_v[pl.ds(16 * r, 16)] = acc
        pltpu.sync_copy(eob_v, eob_hbm)


_SC_MESH = dict(
    mesh=plsc.VectorSubcoreMesh(core_axis_name="c", subcore_axis_name="s"),
    compiler_params=pltpu.CompilerParams(needs_layout_passes=False),
)


def _dispatch(x, idx_flat):
    counts, ranks = pl.kernel(
        _count_body,
        out_type=[
            jax.ShapeDtypeStruct((NW, 16), jnp.int32),
            jax.ShapeDtypeStruct((N,), jnp.int32),
        ],
        scratch_types=[
            pltpu.VMEM((CH,), jnp.int32),
            pltpu.VMEM((CH,), jnp.int32),
            pltpu.VMEM((16,), jnp.int32),
        ],
        **_SC_MESH,
    )(idx_flat)
    return pl.kernel(
        _dest_body,
        out_type=[
            jax.ShapeDtypeStruct((N // 128, 128), jnp.int32),
            jax.ShapeDtypeStruct((NSLOT, D), jnp.float32),
            jax.ShapeDtypeStruct((EOB_PAD,), jnp.int32),
        ],
        scratch_types=[
            pltpu.VMEM((CH,), jnp.int32),
            pltpu.VMEM((CH,), jnp.int32),
            pltpu.VMEM((CH // 128, 128), jnp.int32),
            pltpu.VMEM((NGC, GCH), jnp.int32),
            pltpu.VMEM((NGC, GCH), jnp.int32),
            pltpu.VMEM((16,), jnp.int32),
            pltpu.VMEM((NW, 16), jnp.int32),
            pltpu.VMEM((EOB_PAD,), jnp.int32),
            pltpu.VMEM((GCH, D), jnp.float32),
            pltpu.VMEM((GCH, D), jnp.float32),
            pltpu.SemaphoreType.DMA,
            pltpu.SemaphoreType.DMA,
            pltpu.SemaphoreType.DMA,
            pltpu.SemaphoreType.DMA,
        ],
        **_SC_MESH,
    )(x, idx_flat, counts, ranks)


def _gmm_body(eob_ref, nblk_ref, xs_ref, w_ref, out_ref):
    @pl.when(pl.program_id(0) < nblk_ref[0])
    def _():
        out_ref[...] = jax.lax.dot_general(
            xs_ref[...], w_ref[0], (((1,), (1,)), ((), ())),
            preferred_element_type=jnp.float32)


def _grouped_matmul(xs, expert_wb, expert_of_block, nblk):
    grid_spec = pltpu.PrefetchScalarGridSpec(
        num_scalar_prefetch=2,
        grid=(NBLK,),
        in_specs=[
            pl.BlockSpec((BM, D), lambda b, eob, nb: (b, 0)),
            pl.BlockSpec((1, D, D), lambda b, eob, nb: (eob[b], 0, 0)),
        ],
        out_specs=pl.BlockSpec((BM, D), lambda b, eob, nb: (b, 0)),
    )
    return pl.pallas_call(
        _gmm_body,
        grid_spec=grid_spec,
        out_shape=jax.ShapeDtypeStruct((NSLOT, D), jnp.float32),
    )(expert_of_block, nblk, xs, expert_wb)


TCH = T // NW
CCH = 32
NCC = TCH // CCH


def _combine_body(ys_hbm, dest_hbm, out_hbm, didx_v, rows_v, out_v):
    cid = lax.axis_index("c")
    sid = lax.axis_index("s")
    wid = sid * NC + cid
    tbase = wid * TCH

    pltpu.sync_copy(dest_hbm.at[pl.ds(wid * 2 * TCH, 2 * TCH)], didx_v)
    for cch in range(NCC):
        pltpu.sync_copy(
            ys_hbm.at[didx_v.at[pl.ds(cch * 2 * CCH, 2 * CCH)]],
            rows_v)

        def _row(r, _):
            for cg in range(D // 16):
                a = rows_v[2 * r, pl.ds(cg * 16, 16)]
                b = rows_v[2 * r + 1, pl.ds(cg * 16, 16)]
                out_v[r, pl.ds(cg * 16, 16)] = a + b
            return 0

        lax.fori_loop(0, CCH, _row, 0)
        pltpu.sync_copy(out_v, out_hbm.at[pl.ds(tbase + cch * CCH, CCH)])


def _combine(ys, dest_flat):
    return pl.kernel(
        _combine_body,
        out_type=jax.ShapeDtypeStruct((T, D), jnp.float32),
        scratch_types=[
            pltpu.VMEM((2 * TCH,), jnp.int32),
            pltpu.VMEM((2 * CCH, D), jnp.float32),
            pltpu.VMEM((CCH, D), jnp.float32),
        ],
        **_SC_MESH,
    )(ys, dest_flat)


def kernel(x, gate_w, expert_w):
    idx, w = _routing(x, gate_w)
    dest2d, xs, eob_pad = _dispatch(x, idx.reshape(N))
    expert_of_block = eob_pad[:NBLK]
    nblk = eob_pad[EOB_PAD - 1:EOB_PAD]
    ys = _grouped_matmul(xs, expert_w, expert_of_block, nblk)
    dest2 = dest2d.reshape(T, K)
    out = (w[:, 0:1] * ys[dest2[:, 0]] + w[:, 1:2] * ys[dest2[:, 1]])
    return out


# speedup vs baseline: 1.0029x; 1.0029x over previous
import jax
import jax.numpy as jnp
from jax import lax
from jax.experimental import pallas as pl
from jax.experimental.pallas import tpu as pltpu
from jax.experimental.pallas import tpu_sc as plsc

E = 8
D = 1024
T = 4096
K = 2
N = T * K
BM = 256
NBLK = (N + E * (BM - 1) + BM - 1) // BM
NSLOT = NBLK * BM
TB = 512

NC = 2
NS = 16
NW = NC * NS
CH = N // NW
NV = CH // 16
GCH = 32
NGC = CH // GCH
EOB_PAD = 48


def _routing_body(x_ref, gw_ref, idx_ref, w_ref):
    logits = jax.lax.dot_general(
        x_ref[...], gw_ref[...], (((1,), (1,)), ((), ())),
        preferred_element_type=jnp.float32)
    cols = jax.lax.broadcasted_iota(jnp.int32, (TB, E), 1)
    big = jnp.int32(E)
    m0 = jnp.max(logits, axis=1, keepdims=True)
    e0 = jnp.min(jnp.where(logits == m0, cols, big), axis=1, keepdims=True)
    neg = jnp.where(cols == e0, -jnp.inf, logits)
    m1 = jnp.max(neg, axis=1, keepdims=True)
    e1 = jnp.min(jnp.where(neg == m1, cols, big), axis=1, keepdims=True)
    t = jnp.exp(m1 - m0)
    w0 = 1.0 / (1.0 + t)
    w1 = 1.0 - w0
    idx_ref[...] = jnp.concatenate([e0, e1], axis=1)
    w_ref[...] = jnp.concatenate([w0, w1], axis=1)


def _routing(x, gate_w):
    return pl.pallas_call(
        _routing_body,
        grid=(T // TB,),
        in_specs=[
            pl.BlockSpec((TB, D), lambda i: (i, 0)),
            pl.BlockSpec((E, D), lambda i: (0, 0)),
        ],
        out_specs=[
            pl.BlockSpec((TB, K), lambda i: (i, 0)),
            pl.BlockSpec((TB, K), lambda i: (i, 0)),
        ],
        out_shape=[
            jax.ShapeDtypeStruct((T, K), jnp.int32),
            jax.ShapeDtypeStruct((T, K), jnp.float32),
        ],
    )(x, gate_w)


def _count_body(idx_hbm, counts_hbm, ranks_hbm, eloc_v, rank_v, cnt_v):
    cid = lax.axis_index("c")
    sid = lax.axis_index("s")
    wid = sid * NC + cid
    base = wid * CH
    lanes = lax.iota(jnp.int32, 16)

    pltpu.sync_copy(idx_hbm.at[pl.ds(base, CH)], eloc_v)
    c = [jnp.int32(0)] * E
    for i in range(NV):
        v = eloc_v[pl.ds(i * 16, 16)]
        d = jnp.zeros((16,), jnp.int32)
        for e in range(E):
            m = v == e
            mi = m.astype(jnp.int32)
            pre = plsc.cumsum(mi)
            d = jnp.where(m, c[e] + pre - 1, d)
            c[e] = c[e] + jnp.sum(mi)
        rank_v[pl.ds(i * 16, 16)] = d
    cvec = jnp.zeros((16,), jnp.int32)
    for e in range(E):
        cvec = jnp.where(lanes == e, c[e], cvec)
    cnt_v[...] = cvec
    pltpu.sync_copy(cnt_v, counts_hbm.at[wid])
    pltpu.sync_copy(rank_v, ranks_hbm.at[pl.ds(base, CH)])


def _dest_body(x_hbm, idx_hbm, counts_hbm, ranks_hbm,
               dest_hbm, xs_hbm, eob_hbm,
               eloc_v, rank_v, dstrow_v, dsc_v, tok_v, cnt_v, allcnt_v,
               eob_v, rows_v, rows2_v, gsem0, gsem1, ssem0, ssem1):
    cid = lax.axis_index("c")
    sid = lax.axis_index("s")
    wid = sid * NC + cid
    base = wid * CH
    lanes = lax.iota(jnp.int32, 16)

    pltpu.sync_copy(idx_hbm.at[pl.ds(base, CH)], eloc_v)
    pltpu.sync_copy(ranks_hbm.at[pl.ds(base, CH)], rank_v)
    pltpu.sync_copy(counts_hbm, allcnt_v)
    prior = jnp.zeros((16,), jnp.int32)
    gc = jnp.zeros((16,), jnp.int32)
    wid_v = jnp.full((16,), wid, jnp.int32)
    for w in range(NW):
        row = allcnt_v[w]
        gc = gc + row
        prior = prior + jnp.where(wid_v > w, row, 0)
    blocks = (gc + (BM - 1)) // BM
    psum = plsc.cumsum(blocks * BM)
    table = (psum - blocks * BM) + prior
    cnt_v[...] = table
    for i in range(NV):
        v = eloc_v[pl.ds(i * 16, 16)]
        b = plsc.load_gather(cnt_v, [v])
        dv = b + rank_v[pl.ds(i * 16, 16)]
        dstrow_v[(i * 16) // 128, pl.ds((i * 16) % 128, 16)] = dv
        nvc = GCH // 16
        dsc_v[i // nvc, pl.ds((i % nvc) * 16, 16)] = dv
        tok_v[i // nvc, pl.ds((i % nvc) * 16, 16)] = (base + i * 16 + lanes) // K
    for j in range(CH // 128):
        pltpu.sync_copy(dstrow_v.at[j], dest_hbm.at[wid * (CH // 128) + j])
    bufs = (rows_v, rows2_v)
    gsems = (gsem0, gsem1)
    ssems = (ssem0, ssem1)
    g_next = pltpu.async_copy(x_hbm.at[tok_v.at[0]], bufs[0], gsems[0])
    sc_h = [None, None]
    for cch in range(NGC):
        b = cch % 2
        g_next.wait()
        sc_h[b] = pltpu.async_copy(bufs[b], xs_hbm.at[dsc_v.at[cch]], ssems[b])
        if cch + 1 < NGC:
            nb = (cch + 1) % 2
            if sc_h[nb] is not None:
                sc_h[nb].wait()
            g_next = pltpu.async_copy(x_hbm.at[tok_v.at[cch + 1]],
                                      bufs[nb], gsems[nb])
    sc_h[0].wait()
    sc_h[1].wait()

    @pl.when(wid == 0)
    def _eob():
        cbv = plsc.cumsum(blocks)
        for r in range(EOB_PAD // 16):
            bi = lanes + 16 * r
            acc = jnp.zeros((16,), jnp.int32)
            for e in range(E - 1):
                acc = acc + (bi >= cbv[e]).astype(jnp.int32)
            if r == EOB_PAD // 16 - 1:
                acc = jnp.where(lanes == 15, cbv[E - 1], acc)
            eob_v[pl.ds(16 * r, 16)] = acc
        pltpu.sync_copy(eob_v, eob_hbm)


_SC_MESH = dict(
    mesh=plsc.VectorSubcoreMesh(core_axis_name="c", subcore_axis_name="s"),
    compiler_params=pltpu.CompilerParams(needs_layout_passes=False),
)


def _dispatch(x, idx_flat):
    counts, ranks = pl.kernel(
        _count_body,
        out_type=[
            jax.ShapeDtypeStruct((NW, 16), jnp.int32),
            jax.ShapeDtypeStruct((N,), jnp.int32),
        ],
        scratch_types=[
            pltpu.VMEM((CH,), jnp.int32),
            pltpu.VMEM((CH,), jnp.int32),
            pltpu.VMEM((16,), jnp.int32),
        ],
        **_SC_MESH,
    )(idx_flat)
    return pl.kernel(
        _dest_body,
        out_type=[
            jax.ShapeDtypeStruct((N // 128, 128), jnp.int32),
            jax.ShapeDtypeStruct((NSLOT, D), jnp.float32),
            jax.ShapeDtypeStruct((EOB_PAD,), jnp.int32),
        ],
        scratch_types=[
            pltpu.VMEM((CH,), jnp.int32),
            pltpu.VMEM((CH,), jnp.int32),
            pltpu.VMEM((CH // 128, 128), jnp.int32),
            pltpu.VMEM((NGC, GCH), jnp.int32),
            pltpu.VMEM((NGC, GCH), jnp.int32),
            pltpu.VMEM((16,), jnp.int32),
            pltpu.VMEM((NW, 16), jnp.int32),
            pltpu.VMEM((EOB_PAD,), jnp.int32),
            pltpu.VMEM((GCH, D), jnp.float32),
            pltpu.VMEM((GCH, D), jnp.float32),
            pltpu.SemaphoreType.DMA,
            pltpu.SemaphoreType.DMA,
            pltpu.SemaphoreType.DMA,
            pltpu.SemaphoreType.DMA,
        ],
        **_SC_MESH,
    )(x, idx_flat, counts, ranks)


def _gmm_body(eob_ref, nblk_ref, xs_ref, w_ref, out_ref):
    @pl.when(pl.program_id(0) < nblk_ref[0])
    def _():
        out_ref[...] = jax.lax.dot_general(
            xs_ref[...], w_ref[0], (((1,), (1,)), ((), ())),
            precision=jax.lax.Precision.DEFAULT,
            preferred_element_type=jnp.float32)


def _grouped_matmul(xs, expert_wb, expert_of_block, nblk):
    grid_spec = pltpu.PrefetchScalarGridSpec(
        num_scalar_prefetch=2,
        grid=(NBLK,),
        in_specs=[
            pl.BlockSpec((BM, D), lambda b, eob, nb: (b, 0)),
            pl.BlockSpec((1, D, D), lambda b, eob, nb: (eob[b], 0, 0)),
        ],
        out_specs=pl.BlockSpec((BM, D), lambda b, eob, nb: (b, 0)),
    )
    return pl.pallas_call(
        _gmm_body,
        grid_spec=grid_spec,
        out_shape=jax.ShapeDtypeStruct((NSLOT, D), jnp.float32),
    )(expert_of_block, nblk, xs, expert_wb)


TCH = T // NW
CCH = 32
NCC = TCH // CCH


def _combine_body(ys_hbm, dest_hbm, out_hbm, didx_v, rows_v, out_v):
    cid = lax.axis_index("c")
    sid = lax.axis_index("s")
    wid = sid * NC + cid
    tbase = wid * TCH

    pltpu.sync_copy(dest_hbm.at[pl.ds(wid * 2 * TCH, 2 * TCH)], didx_v)
    for cch in range(NCC):
        pltpu.sync_copy(
            ys_hbm.at[didx_v.at[pl.ds(cch * 2 * CCH, 2 * CCH)]],
            rows_v)

        def _row(r, _):
            for cg in range(D // 16):
                a = rows_v[2 * r, pl.ds(cg * 16, 16)]
                b = rows_v[2 * r + 1, pl.ds(cg * 16, 16)]
                out_v[r, pl.ds(cg * 16, 16)] = a + b
            return 0

        lax.fori_loop(0, CCH, _row, 0)
        pltpu.sync_copy(out_v, out_hbm.at[pl.ds(tbase + cch * CCH, CCH)])


def _combine(ys, dest_flat):
    return pl.kernel(
        _combine_body,
        out_type=jax.ShapeDtypeStruct((T, D), jnp.float32),
        scratch_types=[
            pltpu.VMEM((2 * TCH,), jnp.int32),
            pltpu.VMEM((2 * CCH, D), jnp.float32),
            pltpu.VMEM((CCH, D), jnp.float32),
        ],
        **_SC_MESH,
    )(ys, dest_flat)


def kernel(x, gate_w, expert_w):
    idx, w = _routing(x, gate_w)
    dest2d, xs, eob_pad = _dispatch(x, idx.reshape(N))
    expert_of_block = eob_pad[:NBLK]
    nblk = eob_pad[EOB_PAD - 1:EOB_PAD]
    ys = _grouped_matmul(xs, expert_w, expert_of_block, nblk)
    dest2 = dest2d.reshape(T, K)
    out = (w[:, 0:1] * ys[dest2[:, 0]] + w[:, 1:2] * ys[dest2[:, 1]])
    return out


# speedup vs baseline: 1.0034x; 1.0005x over previous
import jax
import jax.numpy as jnp
from jax import lax
from jax.experimental import pallas as pl
from jax.experimental.pallas import tpu as pltpu
from jax.experimental.pallas import tpu_sc as plsc

E = 8
D = 1024
T = 4096
K = 2
N = T * K
BM = 256
NBLK = (N + E * (BM - 1) + BM - 1) // BM
NSLOT = NBLK * BM
TB = 512

NC = 2
NS = 16
NW = NC * NS
CH = N // NW
NV = CH // 16
GCH = 32
NGC = CH // GCH
EOB_PAD = 48


def _routing_body(x_ref, gw_ref, idx_ref, w_ref):
    logits = jax.lax.dot_general(
        x_ref[...], gw_ref[...], (((1,), (1,)), ((), ())),
        preferred_element_type=jnp.float32)
    cols = jax.lax.broadcasted_iota(jnp.int32, (TB, E), 1)
    big = jnp.int32(E)
    m0 = jnp.max(logits, axis=1, keepdims=True)
    e0 = jnp.min(jnp.where(logits == m0, cols, big), axis=1, keepdims=True)
    neg = jnp.where(cols == e0, -jnp.inf, logits)
    m1 = jnp.max(neg, axis=1, keepdims=True)
    e1 = jnp.min(jnp.where(neg == m1, cols, big), axis=1, keepdims=True)
    t = jnp.exp(m1 - m0)
    w0 = 1.0 / (1.0 + t)
    w1 = 1.0 - w0
    idx_ref[...] = jnp.concatenate([e0, e1], axis=1)
    w_ref[...] = jnp.concatenate([w0, w1], axis=1)


def _routing(x, gate_w):
    return pl.pallas_call(
        _routing_body,
        grid=(T // TB,),
        in_specs=[
            pl.BlockSpec((TB, D), lambda i: (i, 0)),
            pl.BlockSpec((E, D), lambda i: (0, 0)),
        ],
        out_specs=[
            pl.BlockSpec((TB, K), lambda i: (i, 0)),
            pl.BlockSpec((TB, K), lambda i: (i, 0)),
        ],
        out_shape=[
            jax.ShapeDtypeStruct((T, K), jnp.int32),
            jax.ShapeDtypeStruct((T, K), jnp.float32),
        ],
    )(x, gate_w)


def _count_body(idx_hbm, counts_hbm, ranks_hbm, eloc_v, rank_v, cnt_v):
    cid = lax.axis_index("c")
    sid = lax.axis_index("s")
    wid = sid * NC + cid
    base = wid * CH
    lanes = lax.iota(jnp.int32, 16)

    pltpu.sync_copy(idx_hbm.at[pl.ds(base, CH)], eloc_v)
    c = [jnp.int32(0)] * E
    for i in range(NV):
        v = eloc_v[pl.ds(i * 16, 16)]
        d = jnp.zeros((16,), jnp.int32)
        for e in range(E):
            m = v == e
            mi = m.astype(jnp.int32)
            pre = plsc.cumsum(mi)
            d = jnp.where(m, c[e] + pre - 1, d)
            c[e] = c[e] + jnp.sum(mi)
        rank_v[pl.ds(i * 16, 16)] = d
    cvec = jnp.zeros((16,), jnp.int32)
    for e in range(E):
        cvec = jnp.where(lanes == e, c[e], cvec)
    cnt_v[...] = cvec
    pltpu.sync_copy(cnt_v, counts_hbm.at[wid])
    pltpu.sync_copy(rank_v, ranks_hbm.at[pl.ds(base, CH)])


def _dest_body(x_hbm, idx_hbm, counts_hbm, ranks_hbm,
               dest_hbm, xs_hbm, eob_hbm,
               eloc_v, rank_v, dstrow_v, dsc_v, tok_v, cnt_v, allcnt_v,
               eob_v, rows_v, rows2_v, gsem0, gsem1, ssem0, ssem1):
    cid = lax.axis_index("c")
    sid = lax.axis_index("s")
    wid = sid * NC + cid
    base = wid * CH
    lanes = lax.iota(jnp.int32, 16)

    pltpu.sync_copy(idx_hbm.at[pl.ds(base, CH)], eloc_v)
    pltpu.sync_copy(ranks_hbm.at[pl.ds(base, CH)], rank_v)
    pltpu.sync_copy(counts_hbm, allcnt_v)
    prior = jnp.zeros((16,), jnp.int32)
    gc = jnp.zeros((16,), jnp.int32)
    wid_v = jnp.full((16,), wid, jnp.int32)
    for w in range(NW):
        row = allcnt_v[w]
        gc = gc + row
        prior = prior + jnp.where(wid_v > w, row, 0)
    blocks = (gc + (BM - 1)) // BM
    psum = plsc.cumsum(blocks * BM)
    table = (psum - blocks * BM) + prior
    cnt_v[...] = table
    for i in range(NV):
        v = eloc_v[pl.ds(i * 16, 16)]
        b = plsc.load_gather(cnt_v, [v])
        dv = b + rank_v[pl.ds(i * 16, 16)]
        dstrow_v[(i * 16) // 128, pl.ds((i * 16) % 128, 16)] = dv
        nvc = GCH // 16
        dsc_v[i // nvc, pl.ds((i % nvc) * 16, 16)] = dv
        tok_v[i // nvc, pl.ds((i % nvc) * 16, 16)] = (base + i * 16 + lanes) // K
    for j in range(CH // 128):
        pltpu.sync_copy(dstrow_v.at[j], dest_hbm.at[wid * (CH // 128) + j])
    bufs = (rows_v, rows2_v)
    gsems = (gsem0, gsem1)
    ssems = (ssem0, ssem1)
    g_next = pltpu.async_copy(x_hbm.at[tok_v.at[0]], bufs[0], gsems[0])
    sc_h = [None, None]
    for cch in range(NGC):
        b = cch % 2
        g_next.wait()
        sc_h[b] = pltpu.async_copy(bufs[b], xs_hbm.at[dsc_v.at[cch]], ssems[b])
        if cch + 1 < NGC:
            nb = (cch + 1) % 2
            if sc_h[nb] is not None:
                sc_h[nb].wait()
            g_next = pltpu.async_copy(x_hbm.at[tok_v.at[cch + 1]],
                                      bufs[nb], gsems[nb])
    sc_h[0].wait()
    sc_h[1].wait()

    @pl.when(wid == 0)
    def _eob():
        cbv = plsc.cumsum(blocks)
        for r in range(EOB_PAD // 16):
            bi = lanes + 16 * r
            acc = jnp.zeros((16,), jnp.int32)
            for e in range(E - 1):
                acc = acc + (bi >= cbv[e]).astype(jnp.int32)
            if r == EOB_PAD // 16 - 1:
                acc = jnp.where(lanes == 15, cbv[E - 1], acc)
            eob_v[pl.ds(16 * r, 16)] = acc
        pltpu.sync_copy(eob_v, eob_hbm)


_SC_MESH = dict(
    mesh=plsc.VectorSubcoreMesh(core_axis_name="c", subcore_axis_name="s"),
    compiler_params=pltpu.CompilerParams(needs_layout_passes=False),
)


def _dispatch(x, idx_flat):
    counts, ranks = pl.kernel(
        _count_body,
        out_type=[
            jax.ShapeDtypeStruct((NW, 16), jnp.int32),
            jax.ShapeDtypeStruct((N,), jnp.int32),
        ],
        scratch_types=[
            pltpu.VMEM((CH,), jnp.int32),
            pltpu.VMEM((CH,), jnp.int32),
            pltpu.VMEM((16,), jnp.int32),
        ],
        **_SC_MESH,
    )(idx_flat)
    return pl.kernel(
        _dest_body,
        out_type=[
            jax.ShapeDtypeStruct((N // 128, 128), jnp.int32),
            jax.ShapeDtypeStruct((NSLOT, D), jnp.float32),
            jax.ShapeDtypeStruct((EOB_PAD,), jnp.int32),
        ],
        scratch_types=[
            pltpu.VMEM((CH,), jnp.int32),
            pltpu.VMEM((CH,), jnp.int32),
            pltpu.VMEM((CH // 128, 128), jnp.int32),
            pltpu.VMEM((NGC, GCH), jnp.int32),
            pltpu.VMEM((NGC, GCH), jnp.int32),
            pltpu.VMEM((16,), jnp.int32),
            pltpu.VMEM((NW, 16), jnp.int32),
            pltpu.VMEM((EOB_PAD,), jnp.int32),
            pltpu.VMEM((GCH, D), jnp.float32),
            pltpu.VMEM((GCH, D), jnp.float32),
            pltpu.SemaphoreType.DMA,
            pltpu.SemaphoreType.DMA,
            pltpu.SemaphoreType.DMA,
            pltpu.SemaphoreType.DMA,
        ],
        **_SC_MESH,
    )(x, idx_flat, counts, ranks)


def _gmm_body(eob_ref, nblk_ref, xs_ref, w_ref, out_ref):
    @pl.when(pl.program_id(0) < nblk_ref[0])
    def _():
        out_ref[...] = jax.lax.dot_general(
            xs_ref[...], w_ref[0], (((1,), (1,)), ((), ())),
            preferred_element_type=jnp.float32)


def _grouped_matmul(xs, expert_wb, expert_of_block, nblk):
    grid_spec = pltpu.PrefetchScalarGridSpec(
        num_scalar_prefetch=2,
        grid=(NBLK,),
        in_specs=[
            pl.BlockSpec((BM, D), lambda b, eob, nb: (b, 0)),
            pl.BlockSpec((1, D, D), lambda b, eob, nb: (eob[b], 0, 0)),
        ],
        out_specs=pl.BlockSpec((BM, D), lambda b, eob, nb: (b, 0)),
    )
    return pl.pallas_call(
        _gmm_body,
        grid_spec=grid_spec,
        out_shape=jax.ShapeDtypeStruct((NSLOT, D), jnp.float32),
    )(expert_of_block, nblk, xs, expert_wb)


TCH = T // NW
CCH = 32
NCC = TCH // CCH


def _combine_body(ys_hbm, dest_hbm, out_hbm, didx_v, rows_v, out_v):
    cid = lax.axis_index("c")
    sid = lax.axis_index("s")
    wid = sid * NC + cid
    tbase = wid * TCH

    pltpu.sync_copy(dest_hbm.at[pl.ds(wid * 2 * TCH, 2 * TCH)], didx_v)
    for cch in range(NCC):
        pltpu.sync_copy(
            ys_hbm.at[didx_v.at[pl.ds(cch * 2 * CCH, 2 * CCH)]],
            rows_v)

        def _row(r, _):
            for cg in range(D // 16):
                a = rows_v[2 * r, pl.ds(cg * 16, 16)]
                b = rows_v[2 * r + 1, pl.ds(cg * 16, 16)]
                out_v[r, pl.ds(cg * 16, 16)] = a + b
            return 0

        lax.fori_loop(0, CCH, _row, 0)
        pltpu.sync_copy(out_v, out_hbm.at[pl.ds(tbase + cch * CCH, CCH)])


def _combine(ys, dest_flat):
    return pl.kernel(
        _combine_body,
        out_type=jax.ShapeDtypeStruct((T, D), jnp.float32),
        scratch_types=[
            pltpu.VMEM((2 * TCH,), jnp.int32),
            pltpu.VMEM((2 * CCH, D), jnp.float32),
            pltpu.VMEM((CCH, D), jnp.float32),
        ],
        **_SC_MESH,
    )(ys, dest_flat)


def kernel(x, gate_w, expert_w):
    idx, w = _routing(x, gate_w)
    dest2d, xs, eob_pad = _dispatch(x, idx.reshape(N))
    expert_of_block = eob_pad[:NBLK]
    nblk = eob_pad[EOB_PAD - 1:EOB_PAD]
    ys = _grouped_matmul(xs, expert_w, expert_of_block, nblk)
    dest2 = dest2d.reshape(T, K)
    out = (w[:, 0:1] * ys[dest2[:, 0]] + w[:, 1:2] * ys[dest2[:, 1]])
    return out
